# baseline (speedup 1.0000x reference)
import jax
import jax.numpy as jnp
from jax.experimental import pallas as pl
from jax.experimental.pallas import tpu as pltpu

B = 512
DIM = 128
G = 32
NB = B // G


def _dot(a, b, dims):
    return jax.lax.dot_general(a, b, (dims, ((), ())),
                               preferred_element_type=jnp.float32)


def _iota2(shape, d):
    return jax.lax.broadcasted_iota(jnp.int32, shape, d)


def _gat_block(h, cs, W, a_src, a_dst, b, pe, po, n):
    R = G * n
    xl = _dot(h, W, ((1,), (1,)))
    ad = _dot(xl, a_dst, ((1,), (1,)))
    as_row = _dot(a_src, xl, ((1,), (1,)))
    eye = (_iota2((n, n), 0) == _iota2((n, n), 1)).astype(jnp.float32)
    outs = []
    for g in range(G):
        sl = slice(g * n, (g + 1) * n)
        e = ad[sl, :] + as_row[:, sl]
        e = jnp.where(e > 0.0, e, 0.2 * e)
        cg = cs[g] + eye
        m = jnp.max(e, axis=1, keepdims=True)
        ex = jnp.exp(e - m) * cg
        den = jnp.sum(ex, axis=1, keepdims=True)
        num = _dot(ex, xl[sl, :], ((1,), (0,)))
        outs.append(jnp.maximum(num / den + b, 0.0))
    out = jnp.concatenate(outs, axis=0)
    return jnp.maximum(_dot(pe[:R // 2, :R], out, ((1,), (0,))),
                       _dot(po[:R // 2, :R], out, ((1,), (0,))))


def _pool_masks(cs, n):
    n2 = n // 2
    pr = ((_iota2((n2, n), 1) // 2) == _iota2((n2, n), 0)).astype(jnp.float32)
    pc = ((_iota2((n, n2), 0) // 2) == _iota2((n, n2), 1)).astype(jnp.float32)
    offdiag = (_iota2((n2, n2), 0) != _iota2((n2, n2), 1))
    out = []
    for c in cs:
        cp = _dot(_dot(pr, c, ((1,), (0,))), pc, ((1,), (0,)))
        out.append(jnp.where((cp > 0.0) & offdiag, 1.0, 0.0
                             ).astype(jnp.bfloat16))
    return out


def _accum_stats(stats_ref, row, hp, is_first):
    upd = jnp.concatenate(
        [jnp.sum(hp, axis=0, keepdims=True),
         jnp.sum(hp * hp, axis=0, keepdims=True)], axis=0)

    @pl.when(is_first)
    def _():
        stats_ref[row:row + 2, :] = upd

    @pl.when(jnp.logical_not(is_first))
    def _():
        stats_ref[row:row + 2, :] = stats_ref[row:row + 2, :] + upd


def _bn(h, stats_ref, row, rows):
    s0 = stats_ref[row:row + 1, :]
    s1 = stats_ref[row + 1:row + 2, :]
    mean = s0 / rows
    var = s1 / rows - mean * mean
    return (h - mean) * jax.lax.rsqrt(var + 1e-5)


def _fused_kernel(x_ref, src_ref, dst_ref,
                  w0_ref, as0_ref, ad0_ref, b0_ref,
                  w1_ref, as1_ref, ad1_ref, b1_ref,
                  w2_ref, as2_ref, ad2_ref, b2_ref,
                  pe_ref, po_ref,
                  out_ref,
                  h1_s, h2_s, h3_s, m2_s, m3_s, stats_s):
    p = pl.program_id(0)
    i = pl.program_id(1)
    is_first = i == 0
    pe = pe_ref[...]
    po = po_ref[...]

    @pl.when(p == 0)
    def _pass0():
        io = _iota2((64, 1024), 0)
        cs = []
        for g in range(G):
            s_oh = (io == jnp.bitwise_and(src_ref[g:g + 1, :], 63)
                    ).astype(jnp.bfloat16)
            d_oh = (io == jnp.bitwise_and(dst_ref[g:g + 1, :], 63)
                    ).astype(jnp.bfloat16)
            cs.append(_dot(d_oh, s_oh, ((1,), (1,))))
        hp = _gat_block(x_ref[...], cs, w0_ref[...], as0_ref[...],
                        ad0_ref[...], b0_ref[...], pe, po, 64)
        h1_s[pl.ds(i * (G * 32), G * 32), :] = hp
        _accum_stats(stats_s, 0, hp, is_first)
        ms = _pool_masks(cs, 64)
        for g in range(G):
            m2_s[pl.ds(i * (G * 32) + g * 32, 32), :] = ms[g]

    @pl.when(p == 1)
    def _pass1():
        h = _bn(h1_s[pl.ds(i * (G * 32), G * 32), :], stats_s, 0,
                float(B * 32))
        cs = [m2_s[pl.ds(i * (G * 32) + g * 32, 32), :].astype(jnp.float32)
              for g in range(G)]
        hp = _gat_block(h, cs, w1_ref[...], as1_ref[...],
                        ad1_ref[...], b1_ref[...], pe, po, 32)
        h2_s[pl.ds(i * (G * 16), G * 16), :] = hp
        _accum_stats(stats_s, 2, hp, is_first)
        ms = _pool_masks(cs, 32)
        for g in range(G):
            m3_s[pl.ds(i * (G * 16) + g * 16, 16), :] = ms[g]

    @pl.when(p == 2)
    def _pass2():
        h = _bn(h2_s[pl.ds(i * (G * 16), G * 16), :], stats_s, 2,
                float(B * 16))
        cs = [m3_s[pl.ds(i * (G * 16) + g * 16, 16), :].astype(jnp.float32)
              for g in range(G)]
        hp = _gat_block(h, cs, w2_ref[...], as2_ref[...],
                        ad2_ref[...], b2_ref[...], pe, po, 16)
        h3_s[pl.ds(i * (G * 8), G * 8), :] = hp
        _accum_stats(stats_s, 4, hp, is_first)

    @pl.when(p == 3)
    def _pass3():
        out_ref[...] = _bn(h3_s[pl.ds(i * (G * 8), G * 8), :], stats_s, 4,
                           float(B * 8))


def _pe_const():
    k = jnp.arange(G * 32)[:, None]
    return (jnp.arange(G * 64)[None, :] == 2 * k).astype(jnp.float32)


def _po_const():
    k = jnp.arange(G * 32)[:, None]
    return (jnp.arange(G * 64)[None, :] == 2 * k + 1).astype(jnp.float32)


def kernel(x, edge_index, W0, a_src0, a_dst0, b0, W1, a_src1, a_dst1, b1,
           W2, a_src2, a_dst2, b2):
    src = edge_index[0].reshape(B, 1024)
    dst = edge_index[1].reshape(B, 1024)

    first = lambda p, i: (jnp.where(p == 0, i, 0), 0)
    const = lambda p, i: (0, 0)
    last = lambda p, i: (jnp.where(p == 3, i, 0), 0)

    out = pl.pallas_call(
        _fused_kernel,
        grid=(4, NB),
        in_specs=[
            pl.BlockSpec((G * 64, DIM), first),
            pl.BlockSpec((G, 1024), first),
            pl.BlockSpec((G, 1024), first),
            pl.BlockSpec((DIM, DIM), const),
            pl.BlockSpec((1, DIM), const),
            pl.BlockSpec((1, DIM), const),
            pl.BlockSpec((1, DIM), const),
            pl.BlockSpec((DIM, DIM), const),
            pl.BlockSpec((1, DIM), const),
            pl.BlockSpec((1, DIM), const),
            pl.BlockSpec((1, DIM), const),
            pl.BlockSpec((DIM, DIM), const),
            pl.BlockSpec((1, DIM), const),
            pl.BlockSpec((1, DIM), const),
            pl.BlockSpec((1, DIM), const),
            pl.BlockSpec((G * 32, G * 64), const),
            pl.BlockSpec((G * 32, G * 64), const),
        ],
        out_specs=pl.BlockSpec((G * 8, DIM), last),
        out_shape=jax.ShapeDtypeStruct((B * 8, DIM), jnp.float32),
        scratch_shapes=[
            pltpu.VMEM((B * 32, DIM), jnp.float32),
            pltpu.VMEM((B * 16, DIM), jnp.float32),
            pltpu.VMEM((B * 8, DIM), jnp.float32),
            pltpu.VMEM((B * 32, 32), jnp.bfloat16),
            pltpu.VMEM((B * 16, 16), jnp.bfloat16),
            pltpu.VMEM((8, DIM), jnp.float32),
        ],
        compiler_params=pltpu.CompilerParams(
            dimension_semantics=("arbitrary", "arbitrary")),
    )(x, src, dst,
      W0, a_src0[None, :], a_dst0[None, :], b0[None, :],
      W1, a_src1[None, :], a_dst1[None, :], b1[None, :],
      W2, a_src2[None, :], a_dst2[None, :], b2[None, :],
      _pe_const(), _po_const())
    return out.reshape(B, 8 * DIM)


# speedup vs baseline: 1.5280x; 1.5280x over previous
import jax
import jax.numpy as jnp
from jax.experimental import pallas as pl
from jax.experimental.pallas import tpu as pltpu

B = 512
DIM = 128
G = 32
NB = B // G


def _dot(a, b, dims):
    return jax.lax.dot_general(a, b, (dims, ((), ())),
                               preferred_element_type=jnp.float32)


def _iota2(shape, d):
    return jax.lax.broadcasted_iota(jnp.int32, shape, d)


def _gat_block(h, cs, W, a_src, a_dst, b, n):
    R = G * n
    xl = _dot(h, W, ((1,), (1,)))
    ad = _dot(xl, a_dst, ((1,), (1,)))
    as_row = _dot(a_src, xl, ((1,), (1,)))
    eye = (_iota2((n, n), 0) == _iota2((n, n), 1)).astype(jnp.float32)
    outs = []
    for g in range(G):
        sl = slice(g * n, (g + 1) * n)
        e = ad[sl, :] + as_row[:, sl]
        e = jnp.where(e > 0.0, e, 0.2 * e)
        cg = cs[g] + eye
        em = jnp.where(cg > 0.0, e, -1e30)
        m = jnp.max(em, axis=1, keepdims=True)
        ex = jnp.exp(em - m) * cg
        den = jnp.sum(ex, axis=1, keepdims=True)
        num = _dot(ex, xl[sl, :], ((1,), (0,)))
        outs.append(jnp.maximum(num / den + b, 0.0))
    out = jnp.concatenate(outs, axis=0)
    return jnp.max(out.reshape(R // 2, 2, DIM), axis=1)


def _pool_masks(cs, n):
    n2 = n // 2
    pr = ((_iota2((n2, n), 1) // 2) == _iota2((n2, n), 0)).astype(jnp.float32)
    pc = ((_iota2((n, n2), 0) // 2) == _iota2((n, n2), 1)).astype(jnp.float32)
    offdiag = (_iota2((n2, n2), 0) != _iota2((n2, n2), 1))
    out = []
    for c in cs:
        cp = _dot(_dot(pr, c, ((1,), (0,))), pc, ((1,), (0,)))
        out.append(jnp.where((cp > 0.0) & offdiag, 1.0, 0.0))
    return out


def _accum_stats(stats_ref, row, hp, is_first):
    upd = jnp.concatenate(
        [jnp.sum(hp, axis=0, keepdims=True),
         jnp.sum(hp * hp, axis=0, keepdims=True)], axis=0)

    @pl.when(is_first)
    def _():
        stats_ref[row:row + 2, :] = upd

    @pl.when(jnp.logical_not(is_first))
    def _():
        stats_ref[row:row + 2, :] = stats_ref[row:row + 2, :] + upd


def _bn(h, stats_ref, row, rows):
    s0 = stats_ref[row:row + 1, :]
    s1 = stats_ref[row + 1:row + 2, :]
    mean = s0 / rows
    var = s1 / rows - mean * mean
    return (h - mean) * jax.lax.rsqrt(var + 1e-5)


def _fused_kernel(x_ref, src_ref, dst_ref,
                  w0_ref, as0_ref, ad0_ref, b0_ref,
                  w1_ref, as1_ref, ad1_ref, b1_ref,
                  w2_ref, as2_ref, ad2_ref, b2_ref,
                  out_ref,
                  h1_s, h2_s, h3_s, m2_s, m3_s, stats_s):
    p = pl.program_id(0)
    i = pl.program_id(1)
    is_first = i == 0

    @pl.when(p == 0)
    def _pass0():
        io = _iota2((64, 1024), 0)
        cs = []
        for g in range(G):
            s_oh = (io == jnp.bitwise_and(src_ref[g:g + 1, :], 63)
                    ).astype(jnp.bfloat16)
            d_oh = (io == jnp.bitwise_and(dst_ref[g:g + 1, :], 63)
                    ).astype(jnp.bfloat16)
            cs.append(_dot(d_oh, s_oh, ((1,), (1,))))
        hp = _gat_block(x_ref[...], cs, w0_ref[...], as0_ref[...],
                        ad0_ref[...], b0_ref[...], 64)
        h1_s[pl.ds(i * (G * 32), G * 32), :] = hp
        _accum_stats(stats_s, 0, hp, is_first)
        ms = _pool_masks(cs, 64)
        for g in range(G):
            m2_s[pl.ds(i * (G * 32) + g * 32, 32), :] = ms[g]

    @pl.when(p == 1)
    def _pass1():
        h = _bn(h1_s[pl.ds(i * (G * 32), G * 32), :], stats_s, 0,
                float(B * 32))
        cs = [m2_s[pl.ds(i * (G * 32) + g * 32, 32), :] for g in range(G)]
        hp = _gat_block(h, cs, w1_ref[...], as1_ref[...],
                        ad1_ref[...], b1_ref[...], 32)
        h2_s[pl.ds(i * (G * 16), G * 16), :] = hp
        _accum_stats(stats_s, 2, hp, is_first)
        ms = _pool_masks(cs, 32)
        for g in range(G):
            m3_s[pl.ds(i * (G * 16) + g * 16, 16), :] = ms[g]

    @pl.when(p == 2)
    def _pass2():
        h = _bn(h2_s[pl.ds(i * (G * 16), G * 16), :], stats_s, 2,
                float(B * 16))
        cs = [m3_s[pl.ds(i * (G * 16) + g * 16, 16), :] for g in range(G)]
        hp = _gat_block(h, cs, w2_ref[...], as2_ref[...],
                        ad2_ref[...], b2_ref[...], 16)
        h3_s[pl.ds(i * (G * 8), G * 8), :] = hp
        _accum_stats(stats_s, 4, hp, is_first)

    @pl.when(p == 3)
    def _pass3():
        out_ref[...] = _bn(h3_s[pl.ds(i * (G * 8), G * 8), :], stats_s, 4,
                           float(B * 8))


def kernel(x, edge_index, W0, a_src0, a_dst0, b0, W1, a_src1, a_dst1, b1,
           W2, a_src2, a_dst2, b2):
    src = edge_index[0].reshape(B, 1024)
    dst = edge_index[1].reshape(B, 1024)

    first = lambda p, i: (jnp.where(p == 0, i, 0), 0)
    const = lambda p, i: (0, 0)
    last = lambda p, i: (jnp.where(p == 3, i, 0), 0)

    out = pl.pallas_call(
        _fused_kernel,
        grid=(4, NB),
        in_specs=[
            pl.BlockSpec((G * 64, DIM), first),
            pl.BlockSpec((G, 1024), first),
            pl.BlockSpec((G, 1024), first),
            pl.BlockSpec((DIM, DIM), const),
            pl.BlockSpec((1, DIM), const),
            pl.BlockSpec((1, DIM), const),
            pl.BlockSpec((1, DIM), const),
            pl.BlockSpec((DIM, DIM), const),
            pl.BlockSpec((1, DIM), const),
            pl.BlockSpec((1, DIM), const),
            pl.BlockSpec((1, DIM), const),
            pl.BlockSpec((DIM, DIM), const),
            pl.BlockSpec((1, DIM), const),
            pl.BlockSpec((1, DIM), const),
            pl.BlockSpec((1, DIM), const),
        ],
        out_specs=pl.BlockSpec((G * 8, DIM), last),
        out_shape=jax.ShapeDtypeStruct((B * 8, DIM), jnp.float32),
        scratch_shapes=[
            pltpu.VMEM((B * 32, DIM), jnp.float32),
            pltpu.VMEM((B * 16, DIM), jnp.float32),
            pltpu.VMEM((B * 8, DIM), jnp.float32),
            pltpu.VMEM((B * 32, 32), jnp.float32),
            pltpu.VMEM((B * 16, 16), jnp.float32),
            pltpu.VMEM((8, DIM), jnp.float32),
        ],
        compiler_params=pltpu.CompilerParams(
            dimension_semantics=("arbitrary", "arbitrary")),
    )(x, src, dst,
      W0, a_src0[None, :], a_dst0[None, :], b0[None, :],
      W1, a_src1[None, :], a_dst1[None, :], b1[None, :],
      W2, a_src2[None, :], a_dst2[None, :], b2[None, :])
    return out.reshape(B, 8 * DIM)
